# Initial kernel scaffold; baseline (speedup 1.0000x reference)
import jax
import jax.numpy as jnp
from jax.experimental import pallas as pl


def kernel(pos_idxs, pos_emb):
    raise NotImplementedError("write your pallas kernel here")


# speedup vs baseline: 1.9815x; 1.9815x over previous
import functools

import jax
import jax.numpy as jnp
from jax import lax
from jax.experimental import pallas as pl
from jax.experimental.pallas import tpu as pltpu
from jax.experimental.pallas import tpu_sc as plsc

_CHUNK = 32


@functools.lru_cache(maxsize=None)
def _make_kernel(B: int, D: int, V: int):
    info = plsc.get_sparse_core_info()
    nc, ns = info.num_cores, info.num_subcores
    nw = nc * ns
    assert B % (8 * nw) == 0
    b_per_w = B // nw
    assert b_per_w % _CHUNK == 0
    n_chunks = b_per_w // _CHUNK
    mesh = plsc.VectorSubcoreMesh(core_axis_name="c", subcore_axis_name="s")

    @functools.partial(
        pl.kernel,
        mesh=mesh,
        out_type=jax.ShapeDtypeStruct((B, D), jnp.float32),
        scratch_types=[
            pltpu.VMEM((b_per_w,), jnp.int32),
            pltpu.VMEM((_CHUNK, D), jnp.float32),
            pltpu.SemaphoreType.DMA,
        ],
    )
    def k(table_hbm, idx_hbm, out_hbm, idx_v, rows_v, sem):
        wid = lax.axis_index("s") * nc + lax.axis_index("c")
        base = wid * b_per_w
        pltpu.sync_copy(idx_hbm.at[pl.ds(base, b_per_w)], idx_v)

        def clamp_body(i, carry):
            v = idx_v[pl.ds(i * 16, 16)]
            idx_v[pl.ds(i * 16, 16)] = jnp.minimum(jnp.maximum(v, 0), V - 1)
            return carry

        lax.fori_loop(0, b_per_w // 16, clamp_body, 0)

        def chunk_body(i, carry):
            pltpu.async_copy(
                table_hbm.at[idx_v.at[pl.ds(i * _CHUNK, _CHUNK)]], rows_v, sem
            ).wait()
            pltpu.sync_copy(rows_v, out_hbm.at[pl.ds(base + i * _CHUNK, _CHUNK)])
            return carry

        lax.fori_loop(0, n_chunks, chunk_body, 0)

    return k


def kernel(pos_idxs, pos_emb):
    bsz, seq = pos_idxs.shape
    v, d = pos_emb.shape
    idx = pos_idxs.reshape(bsz * seq).astype(jnp.int32)
    out = _make_kernel(bsz * seq, d, v)(pos_emb, idx)
    return out.reshape(bsz, seq, d)


# speedup vs baseline: 2.3675x; 1.1948x over previous
import functools

import jax
import jax.numpy as jnp
from jax import lax
from jax.experimental import pallas as pl
from jax.experimental.pallas import tpu as pltpu
from jax.experimental.pallas import tpu_sc as plsc

_C = 16
_NBUF = 4


@functools.lru_cache(maxsize=None)
def _make_kernel(B: int, D: int, V: int):
    info = plsc.get_sparse_core_info()
    nc, ns = info.num_cores, info.num_subcores
    nw = nc * ns
    assert B % (8 * nw) == 0
    b_per_w = B // nw
    n_chunks = b_per_w // _C
    assert b_per_w % _C == 0 and n_chunks % _NBUF == 0 and n_chunks >= 2 * _NBUF
    n_groups = n_chunks // _NBUF
    mesh = plsc.VectorSubcoreMesh(core_axis_name="c", subcore_axis_name="s")

    @functools.partial(
        pl.kernel,
        mesh=mesh,
        out_type=jax.ShapeDtypeStruct((B, D), jnp.float32),
        scratch_types=[
            pltpu.VMEM((b_per_w,), jnp.int32),
            *([pltpu.VMEM((_C, D), jnp.float32)] * _NBUF),
            *([pltpu.SemaphoreType.DMA] * (2 * _NBUF)),
        ],
    )
    def k(table_hbm, idx_hbm, out_hbm, idx_v, *rest):
        bufs = rest[:_NBUF]
        gsem = rest[_NBUF:2 * _NBUF]
        osem = rest[2 * _NBUF:]
        wid = lax.axis_index("s") * nc + lax.axis_index("c")
        base = wid * b_per_w
        pltpu.sync_copy(idx_hbm.at[pl.ds(base, b_per_w)], idx_v)

        def clamp_body(i, carry):
            v = idx_v[pl.ds(i * 16, 16)]
            idx_v[pl.ds(i * 16, 16)] = jnp.minimum(jnp.maximum(v, 0), V - 1)
            return carry

        lax.fori_loop(0, b_per_w // 16, clamp_body, 0)

        def start_gather(i, b):
            pltpu.async_copy(
                table_hbm.at[idx_v.at[pl.ds(i * _C, _C)]], bufs[b], gsem[b]
            )

        def wait_gather(b):
            pltpu.make_async_copy(
                table_hbm.at[idx_v.at[pl.ds(0, _C)]], bufs[b], gsem[b]
            ).wait()

        def start_out(i, b):
            pltpu.async_copy(bufs[b], out_hbm.at[pl.ds(base + i * _C, _C)], osem[b])

        def wait_out(b):
            pltpu.make_async_copy(
                bufs[b], out_hbm.at[pl.ds(base, _C)], osem[b]
            ).wait()

        start_gather(0, 0)
        start_gather(1, 1)
        start_gather(2, 2)
        wait_gather(0)
        start_out(0, 0)
        start_gather(3, 3)
        wait_gather(1)
        start_out(1, 1)

        def group(j, carry):
            for b in range(_NBUF):
                i = j * _NBUF + b
                bo = (b + 2) % _NBUF
                wait_out(b)
                start_gather(i, b)
                wait_gather(bo)
                start_out(i - 2, bo)
            return carry

        lax.fori_loop(1, n_groups, group, 0)

        last = n_chunks - 4
        wait_gather(2)
        start_out(last + 2, 2)
        wait_gather(3)
        start_out(last + 3, 3)
        for b in range(_NBUF):
            wait_out(b)

    return k


def kernel(pos_idxs, pos_emb):
    bsz, seq = pos_idxs.shape
    v, d = pos_emb.shape
    idx = pos_idxs.reshape(bsz * seq).astype(jnp.int32)
    out = _make_kernel(bsz * seq, d, v)(pos_emb, idx)
    return out.reshape(bsz, seq, d)


# speedup vs baseline: 2.3759x; 1.0036x over previous
import functools

import jax
import jax.numpy as jnp
from jax import lax
from jax.experimental import pallas as pl
from jax.experimental.pallas import tpu as pltpu
from jax.experimental.pallas import tpu_sc as plsc

_C = 32
_NBUF = 3


@functools.lru_cache(maxsize=None)
def _make_kernel(B: int, D: int, V: int):
    info = plsc.get_sparse_core_info()
    nc, ns = info.num_cores, info.num_subcores
    nw = nc * ns
    assert B % (8 * nw) == 0
    b_per_w = B // nw
    n_chunks = b_per_w // _C
    assert b_per_w % _C == 0 and (n_chunks - 5) % _NBUF == 0 and n_chunks >= 8
    n_main_groups = (n_chunks - 5) // _NBUF
    mesh = plsc.VectorSubcoreMesh(core_axis_name="c", subcore_axis_name="s")

    @functools.partial(
        pl.kernel,
        mesh=mesh,
        out_type=jax.ShapeDtypeStruct((B, D), jnp.float32),
        scratch_types=[
            pltpu.VMEM((b_per_w,), jnp.int32),
            *([pltpu.VMEM((_C, D), jnp.float32)] * _NBUF),
            *([pltpu.SemaphoreType.DMA] * (2 * _NBUF)),
        ],
    )
    def k(table_hbm, idx_hbm, out_hbm, idx_v, *rest):
        bufs = rest[:_NBUF]
        gsem = rest[_NBUF:2 * _NBUF]
        osem = rest[2 * _NBUF:]
        wid = lax.axis_index("s") * nc + lax.axis_index("c")
        base = wid * b_per_w
        pltpu.sync_copy(idx_hbm.at[pl.ds(base, b_per_w)], idx_v)

        def clamp_body(i, carry):
            v = idx_v[pl.ds(i * 16, 16)]
            idx_v[pl.ds(i * 16, 16)] = jnp.minimum(jnp.maximum(v, 0), V - 1)
            return carry

        lax.fori_loop(0, b_per_w // 16, clamp_body, 0)

        def start_gather(i, b):
            pltpu.async_copy(
                table_hbm.at[idx_v.at[pl.ds(i * _C, _C)]], bufs[b], gsem[b]
            )

        def wait_gather(b):
            pltpu.make_async_copy(
                table_hbm.at[idx_v.at[pl.ds(0, _C)]], bufs[b], gsem[b]
            ).wait()

        def start_out(i, b):
            pltpu.async_copy(bufs[b], out_hbm.at[pl.ds(base + i * _C, _C)], osem[b])

        def wait_out(b):
            pltpu.make_async_copy(
                bufs[b], out_hbm.at[pl.ds(base, _C)], osem[b]
            ).wait()

        start_gather(0, 0)
        start_gather(1, 1)
        start_gather(2, 2)
        wait_gather(0)
        start_out(0, 0)

        def group(j, carry):
            for b in range(_NBUF):
                c = j * _NBUF + b
                bo = (b + 1) % _NBUF
                wait_out(b)
                start_gather(c, b)
                wait_gather(bo)
                start_out(c - 2, bo)
            return carry

        lax.fori_loop(1, n_main_groups + 1, group, 0)

        for c in (n_chunks - 2, n_chunks - 1):
            b = c % _NBUF
            bo = (b + 1) % _NBUF
            wait_out(b)
            start_gather(c, b)
            wait_gather(bo)
            start_out(c - 2, bo)
        for c in (n_chunks - 2, n_chunks - 1):
            b = c % _NBUF
            wait_gather(b)
            start_out(c, b)
        for c in (n_chunks - 3, n_chunks - 2, n_chunks - 1):
            wait_out(c % _NBUF)

    return k


def kernel(pos_idxs, pos_emb):
    bsz, seq = pos_idxs.shape
    v, d = pos_emb.shape
    idx = pos_idxs.reshape(bsz * seq).astype(jnp.int32)
    out = _make_kernel(bsz * seq, d, v)(pos_emb, idx)
    return out.reshape(bsz, seq, d)
